# baseline (speedup 1.0000x reference)
import functools

import jax
import jax.numpy as jnp
from jax import lax
from jax.experimental import pallas as pl
from jax.experimental.pallas import tpu as pltpu
from jax.experimental.pallas import tpu_sc as plsc

N = 10000
E = 320000
H = 128
L = 4
G = 64
VOCAB = 128
NFEAT = 9

NC = 2
NS = 16
NW = NC * NS

NP = 12288
NODES_W = NP // NW
EMB_CHUNKS = NFEAT * (NODES_W // 128)

EP = 327680
EDGE_CHUNKS = EP // NW // 128
PAD_DST = N + 8
NACC = 10112
ROWS_T = NACC // NS

_MESH = plsc.VectorSubcoreMesh(core_axis_name="c", subcore_axis_name="s")
_SC_PARAMS = pltpu.CompilerParams(use_tc_tiling_on_sc=False)


def _fill_rows(ref, nrows, ncols, value):
    v = jnp.full((16,), value, dtype=jnp.float32)

    def body(i, _):
        for t in range(ncols // 16):
            ref[i, pl.ds(t * 16, 16)] = v
        return 0

    lax.fori_loop(0, nrows, body, 0)


@functools.partial(
    pl.kernel,
    out_type=(
        jax.ShapeDtypeStruct((NP, H), jnp.float32),
        jax.ShapeDtypeStruct((NC, NACC, 16), jnp.float32),
    ),
    mesh=_MESH,
    compiler_params=_SC_PARAMS,
    scratch_types=dict(
        hacc=pltpu.VMEM_SHARED((NS * 128, H), jnp.float32),
        dacc=pltpu.VMEM_SHARED((NACC, 16), jnp.float32),
        idxb=pltpu.VMEM((EMB_CHUNKS, 128), jnp.int32),
        iotab=pltpu.VMEM((1, 128), jnp.int32),
        gbuf=pltpu.VMEM((4, 128, H), jnp.float32),
        dstb=pltpu.VMEM((EDGE_CHUNKS, 128), jnp.int32),
        onesb=pltpu.VMEM((128, 16), jnp.float32),
        zbuf=pltpu.VMEM((ROWS_T, 16), jnp.float32),
        gsem=pltpu.SemaphoreType.DMA((4,)),
        ssem=pltpu.SemaphoreType.DMA((4,)),
        dsem=pltpu.SemaphoreType.DMA,
        wsem=pltpu.SemaphoreType.DMA,
    ),
)
def _sc_encode(emb_flat, xidx, dst3, h_out, deg_out,
               hacc, dacc, idxb, iotab, gbuf, dstb, onesb, zbuf,
               gsem, ssem, dsem, wsem):
    c = lax.axis_index("c")
    s = lax.axis_index("s")
    w = c * NS + s

    _fill_rows(onesb, 128, 16, 1.0)
    _fill_rows(zbuf, ROWS_T, 16, 0.0)
    base_iota = lax.iota(jnp.int32, 16)
    for t in range(8):
        iotab[0, pl.ds(t * 16, 16)] = base_iota + (s * 128 + t * 16)

    pltpu.sync_copy(zbuf, dacc.at[pl.ds(s * ROWS_T, ROWS_T)])

    nk = NODES_W // 128
    for fk in range(EMB_CHUNKS):
        f, k = fk // nk, fk % nk
        pltpu.sync_copy(xidx.at[f * (NP // 128) + w * nk + k],
                        idxb.at[pl.ds(fk, 1)])
    pltpu.sync_copy(dst3.at[w], dstb)
    for fk in range(EMB_CHUNKS):
        f = fk // nk
        if f > 0:
            for t in range(8):
                idxb[fk, pl.ds(t * 16, 16)] = (
                    idxb[fk, pl.ds(t * 16, 16)] + f * VOCAB)

    plsc.subcore_barrier()

    def gstart(p, j, b):
        pltpu.async_copy(emb_flat.at[idxb.at[j * nk + p]], gbuf.at[b],
                         gsem.at[b])

    def gwait(p, j, b):
        pltpu.make_async_copy(emb_flat.at[idxb.at[j * nk + p]], gbuf.at[b],
                              gsem.at[b]).wait()

    def sstart(j, b, add):
        pltpu.async_copy(gbuf.at[b], hacc.at[iotab.at[0]], ssem.at[b],
                         add=add)

    def swait(j, b):
        pltpu.make_async_copy(gbuf.at[b], hacc.at[iotab.at[0]],
                              ssem.at[b]).wait()

    def wb(p):
        return pltpu.make_async_copy(
            hacc.at[pl.ds(s * 128, 128)],
            h_out.at[pl.ds(c * (NP // NC) + s * NODES_W + p * 128, 128)],
            wsem)

    for p in range(nk):
        gstart(p, 0, 0)
        gstart(p, 1, 1)
        for j in range(NFEAT):
            b = j % 4
            gwait(p, j, b)
            if j == 0:
                if p > 0:
                    wb(p - 1).wait()
                sstart(j, b, False)
            else:
                sstart(j, b, True)
            if j >= 2:
                swait(j - 2, (b - 2) % 4)
            if j + 2 < NFEAT:
                gstart(p, j + 2, (b + 2) % 4)
        swait(NFEAT - 2, (NFEAT - 2) % 4)
        swait(NFEAT - 1, (NFEAT - 1) % 4)
        wb(p).start()

    def dstart(j):
        pltpu.async_copy(onesb, dacc.at[dstb.at[j]], dsem, add=True)

    def dwait():
        pltpu.make_async_copy(onesb, dacc.at[dstb.at[0]], dsem).wait()

    for j in range(8):
        dstart(j)

    def deg_body(j, _):
        dstart(j + 8)
        dwait()
        return 0

    lax.fori_loop(0, EDGE_CHUNKS - 8, deg_body, 0)
    for _ in range(8):
        dwait()
    plsc.subcore_barrier()

    wb(nk - 1).wait()
    pltpu.sync_copy(dacc.at[pl.ds(s * ROWS_T, ROWS_T)],
                    deg_out.at[c, pl.ds(s * ROWS_T, ROWS_T)])


@functools.partial(
    pl.kernel,
    out_type=jax.ShapeDtypeStruct((NC, NACC, H), jnp.float32),
    mesh=_MESH,
    compiler_params=_SC_PARAMS,
    scratch_types=dict(
        acc=pltpu.VMEM_SHARED((NACC, H), jnp.float32),
        srcb=pltpu.VMEM((EDGE_CHUNKS // 2, 128), jnp.int32),
        dstb=pltpu.VMEM((EDGE_CHUNKS // 2, 128), jnp.int32),
        gbuf=pltpu.VMEM((2, 128, H), jnp.float32),
        gsem=pltpu.SemaphoreType.DMA((2,)),
        ssem=pltpu.SemaphoreType.DMA((2,)),
    ),
)
def _sc_agg(hs, src3, dst3, zrows, acc_out, acc, srcb, dstb, gbuf,
            gsem, ssem):
    c = lax.axis_index("c")
    s = lax.axis_index("s")
    w = c * NS + s
    nh = EDGE_CHUNKS // 2

    for r0 in range(0, ROWS_T, 128):
        rows = min(128, ROWS_T - r0)
        pltpu.sync_copy(zrows.at[pl.ds(0, rows)],
                        acc.at[pl.ds(s * ROWS_T + r0, rows)])
    plsc.subcore_barrier()

    def gstart(j, b):
        pltpu.async_copy(hs.at[srcb.at[j]], gbuf.at[b], gsem.at[b])

    def gwait(j, b):
        pltpu.make_async_copy(hs.at[srcb.at[j]], gbuf.at[b], gsem.at[b]).wait()

    def sstart(j, b):
        pltpu.async_copy(gbuf.at[b], acc.at[dstb.at[j]], ssem.at[b], add=True)

    def swait(j, b):
        pltpu.make_async_copy(gbuf.at[b], acc.at[dstb.at[j]],
                              ssem.at[b]).wait()

    for hh in range(2):
        pltpu.sync_copy(src3.at[w, pl.ds(hh * nh, nh)], srcb)
        pltpu.sync_copy(dst3.at[w, pl.ds(hh * nh, nh)], dstb)

        gstart(0, 0)
        gwait(0, 0)
        sstart(0, 0)
        gstart(1, 1)

        def agg_group(g, _):
            for i in range(2):
                j = 1 + g * 2 + i
                b = (1 + i) % 2
                gwait(j, b)
                sstart(j, b)
                swait(j - 1, 1 - b)
                gstart(j + 1, 1 - b)
            return 0

        lax.fori_loop(0, (nh - 2) // 2, agg_group, 0)
        gwait(nh - 1, (nh - 1) % 2)
        sstart(nh - 1, (nh - 1) % 2)
        swait(nh - 2, (nh - 2) % 2)
        swait(nh - 1, (nh - 1) % 2)
    plsc.subcore_barrier()

    pltpu.sync_copy(acc.at[pl.ds(s * ROWS_T, ROWS_T)],
                    acc_out.at[c, pl.ds(s * ROWS_T, ROWS_T)])


def _tc_prep_body(h_ref, deg2_ref, hs_ref, norm_ref, invdeg_ref):
    deg = deg2_ref[0] + deg2_ref[1] + 1.0
    norm = lax.rsqrt(deg)
    norm_ref[...] = norm
    invdeg_ref[...] = 1.0 / deg
    hs_ref[...] = h_ref[...] * norm


def _tc_layer_body(acc0_ref, acc1_ref, h_ref, norm_ref, invdeg_ref,
                   w_ref, b_ref, gamma_ref, beta_ref, hn_ref, hs_ref):
    h = h_ref[...]
    agg = (acc0_ref[...] + acc1_ref[...]) * norm_ref[...] + h * invdeg_ref[...]
    hp = jnp.dot(agg, w_ref[...], preferred_element_type=jnp.float32) + b_ref[...]
    mean = jnp.mean(hp, axis=0, keepdims=True)
    var = jnp.mean((hp - mean) * (hp - mean), axis=0, keepdims=True)
    hb = (hp - mean) * lax.rsqrt(var + 1e-5) * gamma_ref[...] + beta_ref[...]
    hn = jnp.maximum(hb, 0.0) + h
    hn_ref[...] = hn
    hs_ref[...] = hn * norm_ref[...]


def _tc_final_body(h_ref, bid_ref, w1_ref, b1_ref, w2_ref, b2_ref,
                   w3_ref, b3_ref, out_ref):
    bid = bid_ref[...]
    gids = lax.broadcasted_iota(jnp.int32, (N, G), 1)
    mask = (bid == gids).astype(jnp.float32)
    sums = lax.dot_general(mask, h_ref[...], (((0,), (0,)), ((), ())),
                           preferred_element_type=jnp.float32)
    counts = jnp.sum(mask, axis=0)[:, None]
    pooled = sums / jnp.maximum(counts, 1.0)
    z = jnp.maximum(jnp.dot(pooled, w1_ref[...],
                            preferred_element_type=jnp.float32) + b1_ref[...], 0.0)
    z = jnp.maximum(jnp.dot(z, w2_ref[...],
                            preferred_element_type=jnp.float32) + b2_ref[...], 0.0)
    out_ref[...] = jnp.dot(z, w3_ref[...],
                           preferred_element_type=jnp.float32) + b3_ref[...]


_tc_prep = pl.pallas_call(
    _tc_prep_body,
    out_shape=(
        jax.ShapeDtypeStruct((N, H), jnp.float32),
        jax.ShapeDtypeStruct((N, 1), jnp.float32),
        jax.ShapeDtypeStruct((N, 1), jnp.float32),
    ),
)

_tc_layer = pl.pallas_call(
    _tc_layer_body,
    out_shape=(
        jax.ShapeDtypeStruct((N, H), jnp.float32),
        jax.ShapeDtypeStruct((N, H), jnp.float32),
    ),
)

_tc_final = pl.pallas_call(
    _tc_final_body,
    out_shape=jax.ShapeDtypeStruct((G, 1), jnp.float32),
)


def kernel(x, edge_index, batch_ids, atom_emb, Ws, bs, gammas, betas,
           W1, b1, W2, b2, W3, b3):
    emb_flat = atom_emb.reshape(NFEAT * VOCAB, H)
    xidx = jnp.pad(x.astype(jnp.int32).T,
                   ((0, 0), (0, NP - N))).reshape(NFEAT * NP // 128, 1, 128)
    src = edge_index[0].astype(jnp.int32)
    dst = edge_index[1].astype(jnp.int32)
    src3 = jnp.pad(src, (0, EP - E)).reshape(NW, EDGE_CHUNKS, 128)
    dst3 = jnp.pad(dst, (0, EP - E),
                   constant_values=PAD_DST).reshape(NW, EDGE_CHUNKS, 128)

    zrows = jnp.zeros((128, H), jnp.float32)

    h_pad, deg2 = _sc_encode(emb_flat, xidx, dst3)
    h = h_pad[:N]
    deg2s = deg2[:, :N, :1]

    hs, norm, invdeg = _tc_prep(h, deg2s)

    for i in range(L):
        acc2 = _sc_agg(hs, src3, dst3, zrows)
        h, hs = _tc_layer(acc2[0, :N], acc2[1, :N], h, norm, invdeg,
                          Ws[i], bs[i][None, :], gammas[i][None, :],
                          betas[i][None, :])

    out = _tc_final(h, batch_ids.astype(jnp.int32)[:, None],
                    W1, b1[None, :], W2, b2[None, :], W3, b3[None, :])
    return out


# speedup vs baseline: 1.3052x; 1.3052x over previous
import functools

import jax
import jax.numpy as jnp
from jax import lax
from jax.experimental import pallas as pl
from jax.experimental.pallas import tpu as pltpu
from jax.experimental.pallas import tpu_sc as plsc

N = 10000
E = 320000
H = 128
L = 4
G = 64
VOCAB = 128
NFEAT = 9

NC = 2
NS = 16
NW = NC * NS

NP = 12288
NODES_W = NP // NW
EMB_CHUNKS = NODES_W // 128

EP = 323584
EDGE_CHUNKS = EP // NW // 128
PAD_DST = N + 8
NACC = 10112
ROWS_T = NACC // NS

_MESH = plsc.VectorSubcoreMesh(core_axis_name="c", subcore_axis_name="s")
_SC_PARAMS = pltpu.CompilerParams(use_tc_tiling_on_sc=False)


def _fill_rows(ref, nrows, ncols, value):
    v = jnp.full((16,), value, dtype=jnp.float32)

    def body(i, _):
        for t in range(ncols // 16):
            ref[i, pl.ds(t * 16, 16)] = v
        return 0

    lax.fori_loop(0, nrows, body, 0)


@functools.partial(
    pl.kernel,
    out_type=jax.ShapeDtypeStruct((NP, H), jnp.float32),
    mesh=_MESH,
    compiler_params=_SC_PARAMS,
    scratch_types=dict(
        hacc=pltpu.VMEM_SHARED((NP // NC, H), jnp.float32),
        idxrow=pltpu.VMEM((1, 128), jnp.int32),
        iotab=pltpu.VMEM((EMB_CHUNKS, 128), jnp.int32),
        gtmp=pltpu.VMEM((128, H), jnp.float32),
    ),
)
def _sc_encode(emb_flat, xt_pad, h_out, hacc, idxrow, iotab, gtmp):
    c = lax.axis_index("c")
    s = lax.axis_index("s")
    w = c * NS + s

    base_iota = lax.iota(jnp.int32, 16)
    for k in range(EMB_CHUNKS):
        for t in range(8):
            iotab[k, pl.ds(t * 16, 16)] = base_iota + (s * NODES_W + k * 128 + t * 16)

    for f in range(NFEAT):
        for k in range(EMB_CHUNKS):
            row = f * (NP // 128) + w * EMB_CHUNKS + k
            pltpu.sync_copy(xt_pad.at[row], idxrow)
            for t in range(8):
                idxrow[0, pl.ds(t * 16, 16)] = (
                    idxrow[0, pl.ds(t * 16, 16)] + f * VOCAB)
            pltpu.sync_copy(emb_flat.at[idxrow.at[0]], gtmp)
            pltpu.sync_copy(gtmp, hacc.at[iotab.at[k]], add=(f > 0))

    pltpu.sync_copy(hacc.at[pl.ds(s * NODES_W, NODES_W)],
                    h_out.at[pl.ds((c * NS + s) * NODES_W, NODES_W)])


@functools.partial(
    pl.kernel,
    out_type=jax.ShapeDtypeStruct((NC, NACC, H), jnp.float32),
    mesh=_MESH,
    compiler_params=_SC_PARAMS,
    scratch_types=dict(
        acc=pltpu.VMEM_SHARED((NACC, H), jnp.float32),
        srcb=pltpu.VMEM((EDGE_CHUNKS, 128), jnp.int32),
        dstb=pltpu.VMEM((EDGE_CHUNKS, 128), jnp.int32),
        gtmp=pltpu.VMEM((128, H), jnp.float32),
    ),
)
def _sc_agg(hs, src3, dst3, acc_out, acc, srcb, dstb, gtmp):
    c = lax.axis_index("c")
    s = lax.axis_index("s")
    w = c * NS + s

    _fill_rows(gtmp, 128, H, 0.0)
    for r0 in range(0, ROWS_T, 128):
        rows = min(128, ROWS_T - r0)
        pltpu.sync_copy(gtmp.at[pl.ds(0, rows)],
                        acc.at[pl.ds(s * ROWS_T + r0, rows)])
    plsc.subcore_barrier()

    pltpu.sync_copy(src3.at[w], srcb)
    pltpu.sync_copy(dst3.at[w], dstb)

    def body(j, _):
        pltpu.sync_copy(hs.at[srcb.at[j]], gtmp)
        pltpu.sync_copy(gtmp, acc.at[dstb.at[j]], add=True)
        return 0

    lax.fori_loop(0, EDGE_CHUNKS, body, 0)
    plsc.subcore_barrier()

    pltpu.sync_copy(acc.at[pl.ds(s * ROWS_T, ROWS_T)],
                    acc_out.at[c, pl.ds(s * ROWS_T, ROWS_T)])


EC = 32768
EDP = 10 * EC
QD = 80


def _tc_deg_body(dst_ref, deg_ref):
    g = pl.program_id(0)
    dstc = dst_ref[0]
    q = dstc // 128
    r = dstc - q * 128
    qi = lax.broadcasted_iota(jnp.int32, (QD, EC), 0)
    ri = lax.broadcasted_iota(jnp.int32, (128, EC), 0)
    oq = (q == qi).astype(jnp.float32)
    orr = (r == ri).astype(jnp.float32)
    m = lax.dot_general(oq, orr, (((1,), (1,)), ((), ())),
                        preferred_element_type=jnp.float32)

    @pl.when(g == 0)
    def _():
        deg_ref[...] = jnp.zeros_like(deg_ref)

    deg_ref[...] += m


_tc_deg = pl.pallas_call(
    _tc_deg_body,
    grid=(10,),
    in_specs=[pl.BlockSpec((1, 1, EC), lambda g: (g, 0, 0))],
    out_specs=pl.BlockSpec((QD, 128), lambda g: (0, 0)),
    out_shape=jax.ShapeDtypeStruct((QD, 128), jnp.float32),
)


def _tc_prep_body(h_ref, deg2_ref, hs_ref, norm_ref, invdeg_ref):
    deg = deg2_ref[...] + 1.0
    norm = lax.rsqrt(deg)
    norm_ref[...] = norm
    invdeg_ref[...] = norm * norm
    hs_ref[...] = h_ref[...] * norm


def _tc_layer_body(acc0_ref, acc1_ref, h_ref, norm_ref, invdeg_ref,
                   w_ref, b_ref, gamma_ref, beta_ref, hn_ref, hs_ref):
    h = h_ref[...]
    agg = (acc0_ref[...] + acc1_ref[...]) * norm_ref[...] + h * invdeg_ref[...]
    hp = jnp.dot(agg, w_ref[...], preferred_element_type=jnp.float32) + b_ref[...]
    mean = jnp.mean(hp, axis=0, keepdims=True)
    var = jnp.mean((hp - mean) * (hp - mean), axis=0, keepdims=True)
    hb = (hp - mean) * lax.rsqrt(var + 1e-5) * gamma_ref[...] + beta_ref[...]
    hn = jnp.maximum(hb, 0.0) + h
    hn_ref[...] = hn
    hs_ref[...] = hn * norm_ref[...]


def _tc_final_body(h_ref, bid_ref, w1_ref, b1_ref, w2_ref, b2_ref,
                   w3_ref, b3_ref, out_ref):
    bid = bid_ref[...]
    gids = lax.broadcasted_iota(jnp.int32, (N, G), 1)
    mask = (bid == gids).astype(jnp.float32)
    sums = lax.dot_general(mask, h_ref[...], (((0,), (0,)), ((), ())),
                           preferred_element_type=jnp.float32)
    counts = jnp.sum(mask, axis=0)[:, None]
    pooled = sums / jnp.maximum(counts, 1.0)
    z = jnp.maximum(jnp.dot(pooled, w1_ref[...],
                            preferred_element_type=jnp.float32) + b1_ref[...], 0.0)
    z = jnp.maximum(jnp.dot(z, w2_ref[...],
                            preferred_element_type=jnp.float32) + b2_ref[...], 0.0)
    out_ref[...] = jnp.dot(z, w3_ref[...],
                           preferred_element_type=jnp.float32) + b3_ref[...]


_tc_prep = pl.pallas_call(
    _tc_prep_body,
    out_shape=(
        jax.ShapeDtypeStruct((N, H), jnp.float32),
        jax.ShapeDtypeStruct((N, 1), jnp.float32),
        jax.ShapeDtypeStruct((N, 1), jnp.float32),
    ),
)

_tc_layer = pl.pallas_call(
    _tc_layer_body,
    compiler_params=pltpu.CompilerParams(vmem_limit_bytes=100 * 1024 * 1024),
    out_shape=(
        jax.ShapeDtypeStruct((N, H), jnp.float32),
        jax.ShapeDtypeStruct((N, H), jnp.float32),
    ),
)

_tc_final = pl.pallas_call(
    _tc_final_body,
    out_shape=jax.ShapeDtypeStruct((G, 1), jnp.float32),
)


def kernel(x, edge_index, batch_ids, atom_emb, Ws, bs, gammas, betas,
           W1, b1, W2, b2, W3, b3):
    emb_flat = atom_emb.reshape(NFEAT * VOCAB, H)
    xt_pad = jnp.pad(x.astype(jnp.int32).T,
                     ((0, 0), (0, NP - N))).reshape(NFEAT * NP // 128, 1, 128)
    src = edge_index[0].astype(jnp.int32)
    dst = edge_index[1].astype(jnp.int32)
    src3 = jnp.pad(src, (0, EP - E)).reshape(NW, EDGE_CHUNKS, 128)
    dst3 = jnp.pad(dst, (0, EP - E),
                   constant_values=PAD_DST).reshape(NW, EDGE_CHUNKS, 128)

    dst_tc = jnp.pad(dst, (0, EDP - E),
                     constant_values=NACC - 1).reshape(10, 1, EC)

    h_pad = _sc_encode(emb_flat, xt_pad)
    h = h_pad[:N]
    deg_m = _tc_deg(dst_tc)
    degc = deg_m.reshape(QD * 128)[:N, None]

    hs, norm, invdeg = _tc_prep(h, degc)

    for i in range(L):
        acc2 = _sc_agg(hs, src3, dst3)
        h, hs = _tc_layer(acc2[0, :N], acc2[1, :N], h, norm, invdeg,
                          Ws[i], bs[i][None, :], gammas[i][None, :],
                          betas[i][None, :])

    out = _tc_final(h, batch_ids.astype(jnp.int32)[:, None],
                    W1, b1[None, :], W2, b2[None, :], W3, b3[None, :])
    return out
